# Initial kernel scaffold; baseline (speedup 1.0000x reference)
import jax
import jax.numpy as jnp
from jax.experimental import pallas as pl


def kernel(pred_out, target_mask):
    raise NotImplementedError("write your pallas kernel here")


# speedup vs baseline: 493.0973x; 493.0973x over previous
import functools
import math

import jax
import jax.numpy as jnp
from jax import lax
from jax.experimental import pallas as pl
from jax.experimental.pallas import tpu as pltpu
from jax.experimental.pallas import tpu_sc as plsc

N = 4096
W = 64
C = 3
NB = 4224
QSTR = C * NB
BINS = 4 * QSTR
EPS = 1e-7
L05 = float(math.log(0.5 + EPS))


def _tc1_body(x_ref, t_ref, vals_ref, masks_ref, scal_ref):
    x = x_ref[...]
    t = t_ref[...]
    x0, x1, x2 = x[0:1], x[1:2], x[2:3]
    best = x0
    pm = jnp.zeros_like(t)
    upd = x1 > best
    pm = jnp.where(upd, 1, pm)
    best = jnp.where(upd, x1, best)
    upd = x2 > best
    pm = jnp.where(upd, 2, pm)

    mask1 = pm == 1
    mask2 = pm == 2
    masks_ref[0:1, :] = mask1.astype(jnp.int32)
    masks_ref[1:2, :] = mask2.astype(jnp.int32)

    for i, xv in ((0, x1), (1, x2)):
        s = 1.0 / (1.0 + jnp.exp(-xv))
        vals_ref[3 * i + 0:3 * i + 1, :] = s
        vals_ref[3 * i + 1:3 * i + 2, :] = jnp.log(s + EPS)
        vals_ref[3 * i + 2:3 * i + 3, :] = jnp.log(1.0 - s + EPS)

    bp = jnp.where(pm > 0, x1, 0.0)
    p = 1.0 / (1.0 + jnp.exp(-bp))
    tb = (t > 0).astype(jnp.float32)
    bce = -jnp.sum(tb * jnp.log(p + EPS) + (1.0 - tb) * jnp.log(1.0 - p + EPS)) / N
    inter = jnp.sum(p * tb)
    dice = 1.0 - (2.0 * inter + 1.0) / (jnp.sum(p) + jnp.sum(tb) + 1.0)
    res0 = bce + dice

    t_counts = [jnp.sum((t == j).astype(jnp.float32)) for j in range(C)]
    has1 = jnp.sum(mask1.astype(jnp.float32))
    has2 = jnp.sum(mask2.astype(jnp.float32))

    lane = lax.broadcasted_iota(jnp.int32, (1, 128), 1)
    vec = jnp.where(lane == 0, res0, 0.0)
    for j in range(C):
        vec = vec + jnp.where(lane == 1 + j, t_counts[j], 0.0)
    vec = vec + jnp.where(lane == 4, has1, 0.0) + jnp.where(lane == 5, has2, 0.0)
    scal_ref[...] = vec


_tc1 = pl.pallas_call(
    _tc1_body,
    out_shape=(
        jax.ShapeDtypeStruct((6, N), jnp.float32),
        jax.ShapeDtypeStruct((2, N), jnp.int32),
        jax.ShapeDtypeStruct((1, 128), jnp.float32),
    ),
)


def _sc_body(masks_hbm, tgt_hbm, vals_hbm, bins_hbm,
             lab_v, msk_v, tc_v, nb0, nb1, nb2, nb3, sgh_v, bins_v):
    cid = lax.axis_index("c")
    sid = lax.axis_index("s")

    @pl.when(sid == 0)
    def _work():
        pltpu.sync_copy(masks_hbm.at[cid], msk_v)
        pltpu.sync_copy(tgt_hbm, tc_v)
        pltpu.sync_copy(vals_hbm.at[cid], sgh_v)

        def init_body(c, carry):
            base = c * 16
            ii = base + lax.iota(jnp.int32, 16)
            m = msk_v[pl.ds(base, 16)]
            lab_v[pl.ds(base, 16)] = ii
            col = lax.rem(ii, W)
            selfm = m > 0
            for nbref, d, kind in ((nb0, -W, "u"), (nb1, W, "d"),
                                   (nb2, -1, "l"), (nb3, 1, "r")):
                cand = ii + d
                candc = jnp.clip(cand, 0, N - 1)
                nm = plsc.load_gather(msk_v, [candc])
                if kind == "u":
                    valid = cand >= 0
                elif kind == "d":
                    valid = cand < N
                elif kind == "l":
                    valid = col > 0
                else:
                    valid = col < W - 1
                ok = valid & (nm > 0) & selfm
                nbref[pl.ds(base, 16)] = jnp.where(ok, candc, ii)
            return carry

        lax.fori_loop(0, N // 16, init_body, 0)

        def hook(c, ch):
            base = c * 16
            l0 = lab_v[pl.ds(base, 16)]
            l = l0
            for nbref in (nb0, nb1, nb2, nb3):
                idx = nbref[pl.ds(base, 16)]
                l = jnp.minimum(l, plsc.load_gather(lab_v, [idx]))
            lab_v[pl.ds(base, 16)] = l
            return ch | jnp.any(l != l0).astype(jnp.int32)

        def compress(c, carry):
            base = c * 16
            l = lab_v[pl.ds(base, 16)]
            lab_v[pl.ds(base, 16)] = plsc.load_gather(lab_v, [l])
            return carry

        def cc_round(chg):
            ch = lax.fori_loop(0, N // 16, hook, jnp.int32(0))
            lax.fori_loop(0, N // 16, compress, 0)
            lax.fori_loop(0, N // 16, compress, 0)
            return ch

        lax.while_loop(lambda ch: ch > 0, cc_round, jnp.int32(1))

        def zero(c, carry):
            bins_v[pl.ds(c * 16, 16)] = jnp.zeros((16,), jnp.float32)
            return carry

        lax.fori_loop(0, BINS // 16, zero, 0)

        ones = jnp.ones((16,), jnp.float32)

        def scatter(c, carry):
            base = c * 16
            l = lab_v[pl.ds(base, 16)]
            m = msk_v[pl.ds(base, 16)]
            tc = tc_v[pl.ds(base, 16)]
            lf = jnp.where(m > 0, l, -1)
            b0 = tc * NB + (lf + 1)
            plsc.addupdate_scatter(bins_v, [b0], ones)
            plsc.addupdate_scatter(bins_v, [b0 + QSTR], sgh_v[0, pl.ds(base, 16)])
            plsc.addupdate_scatter(bins_v, [b0 + 2 * QSTR], sgh_v[1, pl.ds(base, 16)])
            plsc.addupdate_scatter(bins_v, [b0 + 3 * QSTR], sgh_v[2, pl.ds(base, 16)])
            return carry

        lax.fori_loop(0, N // 16, scatter, 0)
        pltpu.sync_copy(bins_v, bins_hbm.at[cid])


@functools.cache
def _make_sc():
  return pl.kernel(
    _sc_body,
    out_type=jax.ShapeDtypeStruct((2, BINS), jnp.float32),
    mesh=plsc.VectorSubcoreMesh(core_axis_name="c", subcore_axis_name="s"),
    compiler_params=pltpu.CompilerParams(needs_layout_passes=False),
    scratch_types=[
        pltpu.VMEM((N,), jnp.int32),
        pltpu.VMEM((N,), jnp.int32),
        pltpu.VMEM((N,), jnp.int32),
        pltpu.VMEM((N,), jnp.int32),
        pltpu.VMEM((N,), jnp.int32),
        pltpu.VMEM((N,), jnp.int32),
        pltpu.VMEM((N,), jnp.int32),
        pltpu.VMEM((3, N), jnp.float32),
        pltpu.VMEM((BINS,), jnp.float32),
    ],
  )


def _tc2_body(cnt_ref, a_ref, g_ref, h_ref, scal_ref, out_ref):
    res = scal_ref[0, 0]
    t_tot = [scal_ref[0, 1], scal_ref[0, 2], scal_ref[0, 3]]
    has_v = [scal_ref[0, 4] > 0, scal_ref[0, 5] > 0]
    tp = [t_tot[j] > 0 for j in range(C)]
    lin = lax.broadcasted_iota(jnp.int32, (1, NB), 1)
    inf = jnp.float32(jnp.inf)

    for v in range(2):
        cntv = cnt_ref[v]
        av = a_ref[v]
        gv = g_ref[v]
        hv = h_ref[v]
        n_c = jnp.sum(cntv, axis=0, keepdims=True)
        s_c = jnp.sum(av, axis=0, keepdims=True)
        h_c = jnp.sum(hv, axis=0, keepdims=True)
        pres = n_c > 0
        sump = s_c + 0.5 * (N - n_c)
        lmat = []
        for j in range(C):
            bce_sum = gv[j:j + 1] + (h_c - hv[j:j + 1]) + (N - n_c) * L05
            inter = av[j:j + 1] + 0.5 * (t_tot[j] - cntv[j:j + 1])
            lmat.append(-bce_sum / N + 1.0
                        - (2.0 * inter + 1.0) / (sump + t_tot[j] + 1.0))

        tp_v = list(tp)
        res_v = res
        for k in range(C):
            tpf = [jnp.where(b, 1.0, 0.0) for b in tp_v]
            n_t = tpf[0] + tpf[1] + tpf[2]
            active = jnp.float32(k) < n_t
            c0 = tpf[0]
            c1 = c0 + tpf[1]
            c2 = c1 + tpf[2]
            sel = [tp_v[0] & (c0 - 1.0 == k), tp_v[1] & (c1 - 1.0 == k),
                   tp_v[2] & (c2 - 1.0 == k)]
            lcol = jnp.where(sel[0], lmat[0],
                             jnp.where(sel[1], lmat[1],
                                       jnp.where(sel[2], lmat[2], lmat[0])))
            masked = jnp.where(pres, lcol, inf)
            mval = jnp.min(masked)
            idx = jnp.min(jnp.where(masked == mval, lin, jnp.int32(2**30)))
            matched = active & (mval < 1e37)
            res_v = res_v + jnp.where(matched, mval, 0.0)
            pres = pres & jnp.logical_not(matched & (lin == idx))
            tp_v = [tp_v[j] & jnp.logical_not(matched & sel[j])
                    for j in range(C)]
        res_v = res_v + jnp.sum(jnp.where(pres, 1.0, 0.0))
        res = jnp.where(has_v[v], res_v, res)
        tp = [jnp.where(has_v[v], tp_v[j], tp[j]) for j in range(C)]

    total = res
    for j in range(C):
        total = total + jnp.where(tp[j], 1.0, 0.0)
    out_ref[...] = jnp.reshape(total, (1, 1))


_tc2 = pl.pallas_call(
    _tc2_body,
    out_shape=jax.ShapeDtypeStruct((1, 1), jnp.float32),
)


def kernel(pred_out, target_mask):
    x = pred_out.reshape(C, N)
    t = target_mask.reshape(1, N)
    vals, masks, scal = _tc1(x, t)
    bins = _make_sc()(masks, t.reshape(N), vals.reshape(2, C, N))
    b = bins.reshape(2, 4, C, NB)
    out = _tc2(b[:, 0], b[:, 1], b[:, 2], b[:, 3], scal)
    return out.reshape(())


# speedup vs baseline: 860.0833x; 1.7442x over previous
import functools
import math

import jax
import jax.numpy as jnp
from jax import lax
from jax.experimental import pallas as pl
from jax.experimental.pallas import tpu as pltpu
from jax.experimental.pallas import tpu_sc as plsc

N = 4096
W = 64
C = 3
NB = 4224
QSTR = C * NB
BINS = 4 * QSTR
NCHUNK = N // 16
EPS = 1e-7
L05 = float(math.log(0.5 + EPS))


def _tc1_body(x_ref, t_ref, vals_ref, masks_ref, nbs_ref, scal_ref):
    x = x_ref[...]
    t = t_ref[...]
    x0, x1, x2 = x[0:1], x[1:2], x[2:3]
    best = x0
    pm = jnp.zeros_like(t)
    upd = x1 > best
    pm = jnp.where(upd, 1, pm)
    best = jnp.where(upd, x1, best)
    upd = x2 > best
    pm = jnp.where(upd, 2, pm)

    lane = lax.broadcasted_iota(jnp.int32, (1, N), 1)
    col = lax.rem(lane, W)
    zcol = jnp.zeros((1, 1), jnp.int32)

    for i, v in ((0, 1), (1, 2)):
        maskv = (pm == v).astype(jnp.int32)
        masks_ref[i:i + 1, :] = maskv
        shifts = (
            (jnp.concatenate([zcol.repeat(W, 1), maskv[:, :-W]], 1), -W,
             lane >= W),
            (jnp.concatenate([maskv[:, W:], zcol.repeat(W, 1)], 1), W,
             lane < N - W),
            (jnp.concatenate([zcol, maskv[:, :-1]], 1), -1, col > 0),
            (jnp.concatenate([maskv[:, 1:], zcol], 1), 1, col < W - 1),
        )
        for d, (nm, off, valid) in enumerate(shifts):
            ok = valid & (nm > 0) & (maskv > 0)
            nbs_ref[4 * i + d:4 * i + d + 1, :] = jnp.where(ok, lane + off, lane)

    for i, xv in ((0, x1), (1, x2)):
        s = 1.0 / (1.0 + jnp.exp(-xv))
        vals_ref[3 * i + 0:3 * i + 1, :] = s
        vals_ref[3 * i + 1:3 * i + 2, :] = jnp.log(s + EPS)
        vals_ref[3 * i + 2:3 * i + 3, :] = jnp.log(1.0 - s + EPS)

    bp = jnp.where(pm > 0, x1, 0.0)
    p = 1.0 / (1.0 + jnp.exp(-bp))
    tb = (t > 0).astype(jnp.float32)
    bce = -jnp.sum(tb * jnp.log(p + EPS) + (1.0 - tb) * jnp.log(1.0 - p + EPS)) / N
    inter = jnp.sum(p * tb)
    dice = 1.0 - (2.0 * inter + 1.0) / (jnp.sum(p) + jnp.sum(tb) + 1.0)
    res0 = bce + dice

    t_counts = [jnp.sum((t == j).astype(jnp.float32)) for j in range(C)]
    has1 = jnp.sum((pm == 1).astype(jnp.float32))
    has2 = jnp.sum((pm == 2).astype(jnp.float32))

    sl = lax.broadcasted_iota(jnp.int32, (1, 128), 1)
    vec = jnp.where(sl == 0, res0, 0.0)
    for j in range(C):
        vec = vec + jnp.where(sl == 1 + j, t_counts[j], 0.0)
    vec = vec + jnp.where(sl == 4, has1, 0.0) + jnp.where(sl == 5, has2, 0.0)
    scal_ref[...] = vec


_tc1 = pl.pallas_call(
    _tc1_body,
    out_shape=(
        jax.ShapeDtypeStruct((6, N), jnp.float32),
        jax.ShapeDtypeStruct((2, N), jnp.int32),
        jax.ShapeDtypeStruct((8, N), jnp.int32),
        jax.ShapeDtypeStruct((1, 128), jnp.float32),
    ),
)


def _sc_body(masks_hbm, tgt_hbm, vals_hbm, nbs_hbm, iota_hbm, zeros_hbm,
             bins_hbm, lab_v, msk_v, tc_v, nb_v, sgh_v, bins_v):
    cid = lax.axis_index("c")
    sid = lax.axis_index("s")

    @pl.when(sid == 0)
    def _work():
        pltpu.sync_copy(masks_hbm.at[cid], msk_v)
        pltpu.sync_copy(tgt_hbm, tc_v)
        pltpu.sync_copy(vals_hbm.at[cid], sgh_v)
        pltpu.sync_copy(nbs_hbm.at[cid], nb_v)
        pltpu.sync_copy(iota_hbm, lab_v)
        pltpu.sync_copy(zeros_hbm, bins_v)

        def hook(c, carry):
            ch, rev = carry
            cc = jnp.where(rev > 0, NCHUNK - 1 - c, c)
            base = cc * 16
            l0 = lab_v[pl.ds(base, 16)]
            l = l0
            for d in range(4):
                idx = nb_v[d, pl.ds(base, 16)]
                l = jnp.minimum(l, plsc.load_gather(lab_v, [idx]))
            l = plsc.load_gather(lab_v, [l])
            lab_v[pl.ds(base, 16)] = l
            return (ch | jnp.any(l != l0).astype(jnp.int32), rev)

        def cc_round(carry):
            _, rnd = carry
            ch, _ = lax.fori_loop(0, NCHUNK, hook,
                                  (jnp.int32(0), lax.rem(rnd, 2)))
            return (ch, rnd + 1)

        lax.while_loop(lambda c: c[0] > 0, cc_round,
                       (jnp.int32(1), jnp.int32(0)))

        ones = jnp.ones((16,), jnp.float32)

        def scatter(c, carry):
            for u in range(2):
                base = (2 * c + u) * 16
                l = lab_v[pl.ds(base, 16)]
                m = msk_v[pl.ds(base, 16)]
                tc = tc_v[pl.ds(base, 16)]
                lf = jnp.where(m > 0, l, -1)
                b0 = tc * NB + (lf + 1)
                plsc.addupdate_scatter(bins_v, [b0], ones)
                plsc.addupdate_scatter(bins_v, [b0 + QSTR],
                                       sgh_v[0, pl.ds(base, 16)])
                plsc.addupdate_scatter(bins_v, [b0 + 2 * QSTR],
                                       sgh_v[1, pl.ds(base, 16)])
                plsc.addupdate_scatter(bins_v, [b0 + 3 * QSTR],
                                       sgh_v[2, pl.ds(base, 16)])
            return carry

        lax.fori_loop(0, NCHUNK // 2, scatter, 0)
        pltpu.sync_copy(bins_v, bins_hbm.at[cid])


@functools.cache
def _make_sc():
  return pl.kernel(
    _sc_body,
    out_type=jax.ShapeDtypeStruct((2, BINS), jnp.float32),
    mesh=plsc.VectorSubcoreMesh(core_axis_name="c", subcore_axis_name="s"),
    compiler_params=pltpu.CompilerParams(needs_layout_passes=False),
    scratch_types=[
        pltpu.VMEM((N,), jnp.int32),
        pltpu.VMEM((N,), jnp.int32),
        pltpu.VMEM((N,), jnp.int32),
        pltpu.VMEM((4, N), jnp.int32),
        pltpu.VMEM((3, N), jnp.float32),
        pltpu.VMEM((BINS,), jnp.float32),
    ],
  )


def _tc2_body(cnt_ref, a_ref, g_ref, h_ref, scal_ref, out_ref):
    res = scal_ref[0, 0]
    t_tot = [scal_ref[0, 1], scal_ref[0, 2], scal_ref[0, 3]]
    has_v = [scal_ref[0, 4] > 0, scal_ref[0, 5] > 0]
    tp = [t_tot[j] > 0 for j in range(C)]
    lin = lax.broadcasted_iota(jnp.int32, (1, NB), 1)
    inf = jnp.float32(jnp.inf)

    for v in range(2):
        cntv = cnt_ref[v]
        av = a_ref[v]
        gv = g_ref[v]
        hv = h_ref[v]
        n_c = jnp.sum(cntv, axis=0, keepdims=True)
        s_c = jnp.sum(av, axis=0, keepdims=True)
        h_c = jnp.sum(hv, axis=0, keepdims=True)
        pres = n_c > 0
        sump = s_c + 0.5 * (N - n_c)
        lmat = []
        for j in range(C):
            bce_sum = gv[j:j + 1] + (h_c - hv[j:j + 1]) + (N - n_c) * L05
            inter = av[j:j + 1] + 0.5 * (t_tot[j] - cntv[j:j + 1])
            lmat.append(-bce_sum / N + 1.0
                        - (2.0 * inter + 1.0) / (sump + t_tot[j] + 1.0))

        tp_v = list(tp)
        res_v = res
        for k in range(C):
            tpf = [jnp.where(b, 1.0, 0.0) for b in tp_v]
            n_t = tpf[0] + tpf[1] + tpf[2]
            active = jnp.float32(k) < n_t
            c0 = tpf[0]
            c1 = c0 + tpf[1]
            c2 = c1 + tpf[2]
            sel = [tp_v[0] & (c0 - 1.0 == k), tp_v[1] & (c1 - 1.0 == k),
                   tp_v[2] & (c2 - 1.0 == k)]
            lcol = jnp.where(sel[0], lmat[0],
                             jnp.where(sel[1], lmat[1],
                                       jnp.where(sel[2], lmat[2], lmat[0])))
            masked = jnp.where(pres, lcol, inf)
            mval = jnp.min(masked)
            idx = jnp.min(jnp.where(masked == mval, lin, jnp.int32(2**30)))
            matched = active & (mval < 1e37)
            res_v = res_v + jnp.where(matched, mval, 0.0)
            pres = pres & jnp.logical_not(matched & (lin == idx))
            tp_v = [tp_v[j] & jnp.logical_not(matched & sel[j])
                    for j in range(C)]
        res_v = res_v + jnp.sum(jnp.where(pres, 1.0, 0.0))
        res = jnp.where(has_v[v], res_v, res)
        tp = [jnp.where(has_v[v], tp_v[j], tp[j]) for j in range(C)]

    total = res
    for j in range(C):
        total = total + jnp.where(tp[j], 1.0, 0.0)
    out_ref[...] = jnp.reshape(total, (1, 1))


_tc2 = pl.pallas_call(
    _tc2_body,
    out_shape=jax.ShapeDtypeStruct((1, 1), jnp.float32),
)


def kernel(pred_out, target_mask):
    x = pred_out.reshape(C, N)
    t = target_mask.reshape(1, N)
    vals, masks, nbs, scal = _tc1(x, t)
    bins = _make_sc()(
        masks, t.reshape(N), vals.reshape(2, C, N), nbs.reshape(2, 4, N),
        jnp.arange(N, dtype=jnp.int32), jnp.zeros((BINS,), jnp.float32))
    b = bins.reshape(2, 4, C, NB)
    out = _tc2(b[:, 0], b[:, 1], b[:, 2], b[:, 3], scal)
    return out.reshape(())
